# baseline (speedup 1.0000x reference)
import functools

import jax
import jax.numpy as jnp
from jax import lax
from jax.experimental import pallas as pl
from jax.experimental.pallas import tpu as pltpu
from jax.experimental.pallas import tpu_sc as plsc

NC = 2
NS = 16
L = 16
NW = NC * NS


@functools.lru_cache(maxsize=None)
def _make_pool(B, H, D):
    assert B % NW == 0 and D % L == 0
    b_per_w = B // NW
    mesh = plsc.VectorSubcoreMesh(core_axis_name="c", subcore_axis_name="s")

    @functools.partial(
        pl.kernel,
        out_type=jax.ShapeDtypeStruct((B, D), jnp.float32),
        mesh=mesh,
        scratch_types=[
            pltpu.VMEM((b_per_w * H,), jnp.int32),
            pltpu.VMEM((H, D), jnp.float32),
            pltpu.VMEM((b_per_w, D), jnp.float32),
            pltpu.SemaphoreType.DMA,
        ],
        compiler_params=pltpu.CompilerParams(use_tc_tiling_on_sc=False),
    )
    def pool(x_hbm, table_hbm, out_hbm, idx_v, rows_v, pooled_v, sem):
        wid = lax.axis_index("s") * NC + lax.axis_index("c")
        base = wid * b_per_w
        pltpu.sync_copy(x_hbm.at[pl.ds(base * H, b_per_w * H)], idx_v)

        @pl.loop(0, b_per_w)
        def _sample(i):
            pltpu.async_copy(
                table_hbm.at[idx_v.at[pl.ds(i * H, H)]], rows_v, sem
            ).wait()

            def body(r, accs):
                return tuple(
                    a + rows_v[r, pl.ds(j * L, L)] for j, a in enumerate(accs)
                )

            accs = tuple(jnp.zeros((L,), jnp.float32) for _ in range(D // L))
            accs = lax.fori_loop(0, H, body, accs)
            for j, a in enumerate(accs):
                pooled_v[i, pl.ds(j * L, L)] = a

        pltpu.sync_copy(pooled_v, out_hbm.at[pl.ds(base, b_per_w)])

    return pool


@functools.lru_cache(maxsize=None)
def _make_head(B, D, O, BB=256):
    assert B % BB == 0

    def head(s_ref, wt_ref, b_ref, out_ref):
        logits = (
            jnp.dot(s_ref[...], wt_ref[...], preferred_element_type=jnp.float32)
            + b_ref[...]
        )
        m = jnp.max(logits, axis=-1, keepdims=True)
        e = jnp.exp(logits - m)
        out_ref[...] = e / jnp.sum(e, axis=-1, keepdims=True)

    return pl.pallas_call(
        head,
        grid=(B // BB,),
        in_specs=[
            pl.BlockSpec((BB, D), lambda i: (i, 0)),
            pl.BlockSpec((D, O), lambda i: (0, 0)),
            pl.BlockSpec((1, O), lambda i: (0, 0)),
        ],
        out_specs=pl.BlockSpec((BB, O), lambda i: (i, 0)),
        out_shape=jax.ShapeDtypeStruct((B, O), jnp.float32),
    )


def kernel(x, table, W, b):
    B, H = x.shape
    V, D = table.shape
    O = W.shape[0]
    x_flat = x.reshape(B * H).astype(jnp.int32)
    pooled = _make_pool(B, H, D)(x_flat, table)
    return _make_head(B, D, O)(pooled, W.T, b.reshape(1, O))


# speedup vs baseline: 1.0000x; 1.0000x over previous
import functools

import jax
import jax.numpy as jnp
from jax import lax
from jax.experimental import pallas as pl
from jax.experimental.pallas import tpu as pltpu
from jax.experimental.pallas import tpu_sc as plsc

NC = 2
NS = 16
L = 16
NW = NC * NS


@functools.lru_cache(maxsize=None)
def _make_pool(B, H, D):
    assert B % NW == 0 and D % L == 0
    b_per_w = B // NW
    mesh = plsc.VectorSubcoreMesh(core_axis_name="c", subcore_axis_name="s")

    @functools.partial(
        pl.kernel,
        out_type=jax.ShapeDtypeStruct((B, D), jnp.float32),
        mesh=mesh,
        scratch_types=[
            pltpu.VMEM((b_per_w * H,), jnp.int32),
            pltpu.VMEM((H, D), jnp.float32),
            pltpu.VMEM((b_per_w, D), jnp.float32),
            pltpu.SemaphoreType.DMA,
        ],
        compiler_params=pltpu.CompilerParams(use_tc_tiling_on_sc=False),
    )
    def pool(x_hbm, table_hbm, out_hbm, idx_v, rows_v, pooled_v, sem):
        wid = lax.axis_index("s") * NC + lax.axis_index("c")
        base = wid * b_per_w
        pltpu.sync_copy(x_hbm.at[pl.ds(base * H, b_per_w * H)], idx_v)

        @pl.loop(0, b_per_w)
        def _sample(i):
            pltpu.async_copy(
                table_hbm.at[idx_v.at[pl.ds(i * H, H)]], rows_v, sem
            ).wait()

            def body(r, accs):
                return tuple(
                    a + rows_v[r, pl.ds(j * L, L)] for j, a in enumerate(accs)
                )

            accs = tuple(jnp.zeros((L,), jnp.float32) for _ in range(D // L))
            accs = lax.fori_loop(0, H, body, accs)
            for j, a in enumerate(accs):
                pooled_v[i, pl.ds(j * L, L)] = a

        pltpu.sync_copy(pooled_v, out_hbm.at[pl.ds(base, b_per_w)])

    return pool


@functools.lru_cache(maxsize=None)
def _make_head(B, D, O, BB=256):
    assert B % BB == 0

    def head(s_ref, wt_ref, b_ref, out_ref):
        logits = (
            jnp.dot(s_ref[...], wt_ref[...], preferred_element_type=jnp.float32)
            + b_ref[...]
        )
        m = jnp.max(logits, axis=-1, keepdims=True)
        e = jnp.exp(logits - m)
        out_ref[...] = e / jnp.sum(e, axis=-1, keepdims=True)

    return pl.pallas_call(
        head,
        grid=(B // BB,),
        in_specs=[
            pl.BlockSpec((BB, D), lambda i: (i, 0)),
            pl.BlockSpec((D, O), lambda i: (0, 0)),
            pl.BlockSpec((1, O), lambda i: (0, 0)),
        ],
        out_specs=pl.BlockSpec((BB, O), lambda i: (i, 0)),
        out_shape=jax.ShapeDtypeStruct((B, O), jnp.float32),
    )


def kernel(x, table, W, b):
    B, H = x.shape
    V, D = table.shape
    O = W.shape[0]
    x_flat = x.reshape(B * H).astype(jnp.int32)
    t_lin = jax.lax.optimization_barrier(table.reshape(V * D)).reshape(V, D)
    pooled = _make_pool(B, H, D)(x_flat, t_lin)
    return _make_head(B, D, O)(pooled, W.T, b.reshape(1, O))


# speedup vs baseline: 1.2404x; 1.2404x over previous
import functools

import jax
import jax.numpy as jnp
from jax import lax
from jax.experimental import pallas as pl
from jax.experimental.pallas import tpu as pltpu
from jax.experimental.pallas import tpu_sc as plsc

NC = 2
NS = 16
L = 16
NW = NC * NS


@functools.lru_cache(maxsize=None)
def _make_pool(B, H, D):
    assert B % NW == 0 and D % L == 0
    b_per_w = B // NW
    mesh = plsc.VectorSubcoreMesh(core_axis_name="c", subcore_axis_name="s")

    @functools.partial(
        pl.kernel,
        out_type=jax.ShapeDtypeStruct((B, D), jnp.float32),
        mesh=mesh,
        scratch_types=[
            pltpu.VMEM((b_per_w * H,), jnp.int32),
            pltpu.VMEM((H, D), jnp.float32),
            pltpu.VMEM((H, D), jnp.float32),
            pltpu.VMEM((b_per_w, D), jnp.float32),
            pltpu.SemaphoreType.DMA,
            pltpu.SemaphoreType.DMA,
        ],
        compiler_params=pltpu.CompilerParams(use_tc_tiling_on_sc=False),
    )
    def pool(x_hbm, table_hbm, out_hbm, idx_v, rows0_v, rows1_v, pooled_v,
             sem0, sem1):
        wid = lax.axis_index("s") * NC + lax.axis_index("c")
        base = wid * b_per_w
        pltpu.sync_copy(x_hbm.at[pl.ds(base * H, b_per_w * H)], idx_v)

        def gather(i, buf, sem):
            return pltpu.async_copy(
                table_hbm.at[idx_v.at[pl.ds(i * H, H)]], buf, sem
            )

        def accumulate(i, buf):
            def body(r, accs):
                return tuple(
                    a + buf[r, pl.ds(j * L, L)] for j, a in enumerate(accs)
                )

            accs = tuple(jnp.zeros((L,), jnp.float32) for _ in range(D // L))
            accs = lax.fori_loop(0, H, body, accs, unroll=4)
            for j, a in enumerate(accs):
                pooled_v[i, pl.ds(j * L, L)] = a

        gather(0, rows0_v, sem0)

        @pl.loop(0, b_per_w, step=2)
        def _pair(i):
            gather(i + 1, rows1_v, sem1)
            pltpu.make_async_copy(
                table_hbm.at[idx_v.at[pl.ds(i * H, H)]], rows0_v, sem0
            ).wait()
            accumulate(i, rows0_v)

            @pl.when(i + 2 < b_per_w)
            def _():
                gather(i + 2, rows0_v, sem0)

            pltpu.make_async_copy(
                table_hbm.at[idx_v.at[pl.ds((i + 1) * H, H)]], rows1_v, sem1
            ).wait()
            accumulate(i + 1, rows1_v)

        pltpu.sync_copy(pooled_v, out_hbm.at[pl.ds(base, b_per_w)])

    return pool


@functools.lru_cache(maxsize=None)
def _make_xpose(V, D, CB=2048):
    grid = (pl.cdiv(V, CB),)
    rb = CB * D // 128

    def xpose(t_ref, out_ref):
        tt = t_ref[...].T.reshape(rb, 2, D)
        out_ref[:, 0:D] = tt[:, 0, :]
        out_ref[:, D : 2 * D] = tt[:, 1, :]

    return pl.pallas_call(
        xpose,
        grid=grid,
        in_specs=[pl.BlockSpec((D, CB), lambda i: (0, i))],
        out_specs=pl.BlockSpec((rb, 128), lambda i: (i, 0)),
        out_shape=jax.ShapeDtypeStruct((V * D // 128, 128), jnp.float32),
    )


@functools.lru_cache(maxsize=None)
def _make_head(B, D, O, BB=256):
    assert B % BB == 0

    def head(s_ref, wt_ref, b_ref, out_ref):
        logits = (
            jnp.dot(s_ref[...], wt_ref[...], preferred_element_type=jnp.float32)
            + b_ref[...]
        )
        m = jnp.max(logits, axis=-1, keepdims=True)
        e = jnp.exp(logits - m)
        out_ref[...] = e / jnp.sum(e, axis=-1, keepdims=True)

    return pl.pallas_call(
        head,
        grid=(B // BB,),
        in_specs=[
            pl.BlockSpec((BB, D), lambda i: (i, 0)),
            pl.BlockSpec((D, O), lambda i: (0, 0)),
            pl.BlockSpec((1, O), lambda i: (0, 0)),
        ],
        out_specs=pl.BlockSpec((BB, O), lambda i: (i, 0)),
        out_shape=jax.ShapeDtypeStruct((B, O), jnp.float32),
    )


def kernel(x, table, W, b):
    B, H = x.shape
    V, D = table.shape
    O = W.shape[0]
    x_flat = x.reshape(B * H).astype(jnp.int32)
    t_lin = _make_xpose(V, D)(table.T).reshape(V, D)
    pooled = _make_pool(B, H, D)(x_flat, t_lin)
    return _make_head(B, D, O)(pooled, W.T, b.reshape(1, O))


# speedup vs baseline: 1.4236x; 1.1477x over previous
import functools

import jax
import jax.numpy as jnp
from jax import lax
from jax.experimental import pallas as pl
from jax.experimental.pallas import tpu as pltpu
from jax.experimental.pallas import tpu_sc as plsc

NC = 2
NS = 16
L = 16
NW = NC * NS


@functools.lru_cache(maxsize=None)
def _make_pool(B, H, D):
    assert B % NW == 0 and D % L == 0
    b_per_w = B // NW
    mesh = plsc.VectorSubcoreMesh(core_axis_name="c", subcore_axis_name="s")

    @functools.partial(
        pl.kernel,
        out_type=jax.ShapeDtypeStruct((B, D), jnp.float32),
        mesh=mesh,
        scratch_types=[
            pltpu.VMEM((b_per_w * H,), jnp.int32),
            pltpu.VMEM((H, D), jnp.float32),
            pltpu.VMEM((H, D), jnp.float32),
            pltpu.VMEM((b_per_w, D), jnp.float32),
            pltpu.SemaphoreType.DMA,
            pltpu.SemaphoreType.DMA,
        ],
        compiler_params=pltpu.CompilerParams(use_tc_tiling_on_sc=False),
    )
    def pool(x_hbm, table_hbm, out_hbm, idx_v, rows0_v, rows1_v, pooled_v,
             sem0, sem1):
        wid = lax.axis_index("s") * NC + lax.axis_index("c")
        base = wid * b_per_w
        pltpu.sync_copy(x_hbm.at[pl.ds(base * H, b_per_w * H)], idx_v)

        def gather(i, buf, sem):
            return pltpu.async_copy(
                table_hbm.at[idx_v.at[pl.ds(i * H, H)]], buf, sem
            )

        def accumulate(i, buf):
            def body(r, accs):
                return tuple(
                    a + buf[r, pl.ds(j * L, L)] for j, a in enumerate(accs)
                )

            accs = tuple(jnp.zeros((L,), jnp.float32) for _ in range(D // L))
            accs = lax.fori_loop(0, H, body, accs, unroll=4)
            for j, a in enumerate(accs):
                pooled_v[i, pl.ds(j * L, L)] = a

        gather(0, rows0_v, sem0)

        @pl.loop(0, b_per_w, step=2)
        def _pair(i):
            gather(i + 1, rows1_v, sem1)
            pltpu.make_async_copy(
                table_hbm.at[idx_v.at[pl.ds(i * H, H)]], rows0_v, sem0
            ).wait()
            accumulate(i, rows0_v)

            @pl.when(i + 2 < b_per_w)
            def _():
                gather(i + 2, rows0_v, sem0)

            pltpu.make_async_copy(
                table_hbm.at[idx_v.at[pl.ds((i + 1) * H, H)]], rows1_v, sem1
            ).wait()
            accumulate(i + 1, rows1_v)

        pltpu.sync_copy(pooled_v, out_hbm.at[pl.ds(base, b_per_w)])

    return pool


XPOSE_CB = 2048


@functools.lru_cache(maxsize=None)
def _make_xpose(V, D, CB=XPOSE_CB):
    assert 2 * D == 128
    CB2 = CB // 2
    nblk = pl.cdiv(V, CB)

    def xpose(t_ref, out_ref):
        out_ref[:, 0:D] = t_ref[:, 0:CB2].T
        out_ref[:, D : 2 * D] = t_ref[:, CB2:CB].T

    return pl.pallas_call(
        xpose,
        grid=(nblk,),
        in_specs=[pl.BlockSpec((D, CB), lambda i: (0, i))],
        out_specs=pl.BlockSpec((CB2, 128), lambda i: (i, 0)),
        out_shape=jax.ShapeDtypeStruct((nblk * CB2, 128), jnp.float32),
    )


def _permute_idx(x, CB=XPOSE_CB):
    CB2 = CB // 2
    blk = x // CB
    r = x - blk * CB
    half = r // CB2
    p = r - half * CB2
    return blk * CB + 2 * p + half


@functools.lru_cache(maxsize=None)
def _make_head(B, D, O, BB=256):
    assert B % BB == 0

    def head(s_ref, wt_ref, b_ref, out_ref):
        logits = (
            jnp.dot(s_ref[...], wt_ref[...], preferred_element_type=jnp.float32)
            + b_ref[...]
        )
        m = jnp.max(logits, axis=-1, keepdims=True)
        e = jnp.exp(logits - m)
        out_ref[...] = e / jnp.sum(e, axis=-1, keepdims=True)

    return pl.pallas_call(
        head,
        grid=(B // BB,),
        in_specs=[
            pl.BlockSpec((BB, D), lambda i: (i, 0)),
            pl.BlockSpec((D, O), lambda i: (0, 0)),
            pl.BlockSpec((1, O), lambda i: (0, 0)),
        ],
        out_specs=pl.BlockSpec((BB, O), lambda i: (i, 0)),
        out_shape=jax.ShapeDtypeStruct((B, O), jnp.float32),
    )


def kernel(x, table, W, b):
    B, H = x.shape
    V, D = table.shape
    O = W.shape[0]
    x_flat = _permute_idx(x.reshape(B * H).astype(jnp.int32))
    scratch = _make_xpose(V, D)(table.T)
    Vp = scratch.shape[0] * scratch.shape[1] // D
    t_lin = scratch.reshape(Vp, D)
    pooled = _make_pool(B, H, D)(x_flat, t_lin)
    return _make_head(B, D, O)(pooled, W.T, b.reshape(1, O))


# speedup vs baseline: 2.2017x; 1.5466x over previous
import functools

import jax
import jax.numpy as jnp
from jax import lax
from jax.experimental import pallas as pl
from jax.experimental.pallas import tpu as pltpu
from jax.experimental.pallas import tpu_sc as plsc

NC = 2
NS = 16
L = 16
NW = NC * NS


@functools.lru_cache(maxsize=None)
def _make_pool(B, H, D):
    assert B % NW == 0 and D % L == 0
    b_per_w = B // NW
    mesh = plsc.VectorSubcoreMesh(core_axis_name="c", subcore_axis_name="s")

    @functools.partial(
        pl.kernel,
        out_type=jax.ShapeDtypeStruct((B, D), jnp.float32),
        mesh=mesh,
        scratch_types=[
            pltpu.VMEM((b_per_w * H,), jnp.int32),
            pltpu.VMEM((H, D), jnp.float32),
            pltpu.VMEM((H, D), jnp.float32),
            pltpu.VMEM((b_per_w, D), jnp.float32),
            pltpu.SemaphoreType.DMA,
            pltpu.SemaphoreType.DMA,
        ],
        compiler_params=pltpu.CompilerParams(use_tc_tiling_on_sc=False),
    )
    def pool(x_hbm, table_hbm, out_hbm, idx_v, rows0_v, rows1_v, pooled_v,
             sem0, sem1):
        wid = lax.axis_index("s") * NC + lax.axis_index("c")
        base = wid * b_per_w
        pltpu.sync_copy(x_hbm.at[pl.ds(base * H, b_per_w * H)], idx_v)

        def gather(i, buf, sem):
            return pltpu.async_copy(
                table_hbm.at[idx_v.at[pl.ds(i * H, H)]], buf, sem
            )

        def accumulate(i, buf):
            def body(r, accs):
                return tuple(
                    a + buf[r, pl.ds(j * L, L)] for j, a in enumerate(accs)
                )

            accs = tuple(jnp.zeros((L,), jnp.float32) for _ in range(D // L))
            accs = lax.fori_loop(0, H, body, accs, unroll=4)
            for j, a in enumerate(accs):
                pooled_v[i, pl.ds(j * L, L)] = a

        gather(0, rows0_v, sem0)

        @pl.loop(0, b_per_w, step=2)
        def _pair(i):
            gather(i + 1, rows1_v, sem1)
            pltpu.make_async_copy(
                table_hbm.at[idx_v.at[pl.ds(i * H, H)]], rows0_v, sem0
            ).wait()
            accumulate(i, rows0_v)

            @pl.when(i + 2 < b_per_w)
            def _():
                gather(i + 2, rows0_v, sem0)

            pltpu.make_async_copy(
                table_hbm.at[idx_v.at[pl.ds((i + 1) * H, H)]], rows1_v, sem1
            ).wait()
            accumulate(i + 1, rows1_v)

        pltpu.sync_copy(pooled_v, out_hbm.at[pl.ds(base, b_per_w)])

    return pool


XPOSE_CB = 16384


@functools.lru_cache(maxsize=None)
def _make_xpose(V, D, CB=XPOSE_CB):
    assert 2 * D == 128
    CB2 = CB // 2
    nblk = pl.cdiv(V, CB)

    def xpose(t_ref, out_ref):
        out_ref[:, 0:D] = t_ref[:, 0:CB2].T
        out_ref[:, D : 2 * D] = t_ref[:, CB2:CB].T

    return pl.pallas_call(
        xpose,
        grid=(nblk,),
        in_specs=[pl.BlockSpec((D, CB), lambda i: (0, i))],
        out_specs=pl.BlockSpec((CB2, 128), lambda i: (i, 0)),
        out_shape=jax.ShapeDtypeStruct((nblk * CB2, 128), jnp.float32),
    )


def _permute_idx(x, CB=XPOSE_CB):
    CB2 = CB // 2
    blk = x // CB
    r = x - blk * CB
    half = r // CB2
    p = r - half * CB2
    return blk * CB + 2 * p + half


@functools.lru_cache(maxsize=None)
def _make_head(B, D, O, BB=256):
    assert B % BB == 0

    def head(s_ref, wt_ref, b_ref, out_ref):
        logits = (
            jnp.dot(s_ref[...], wt_ref[...], preferred_element_type=jnp.float32)
            + b_ref[...]
        )
        m = jnp.max(logits, axis=-1, keepdims=True)
        e = jnp.exp(logits - m)
        out_ref[...] = e / jnp.sum(e, axis=-1, keepdims=True)

    return pl.pallas_call(
        head,
        grid=(B // BB,),
        in_specs=[
            pl.BlockSpec((BB, D), lambda i: (i, 0)),
            pl.BlockSpec((D, O), lambda i: (0, 0)),
            pl.BlockSpec((1, O), lambda i: (0, 0)),
        ],
        out_specs=pl.BlockSpec((BB, O), lambda i: (i, 0)),
        out_shape=jax.ShapeDtypeStruct((B, O), jnp.float32),
    )


def kernel(x, table, W, b):
    B, H = x.shape
    V, D = table.shape
    O = W.shape[0]
    x_flat = _permute_idx(x.reshape(B * H).astype(jnp.int32))
    scratch = _make_xpose(V, D)(table.T)
    Vp = scratch.shape[0] * scratch.shape[1] // D
    t_lin = scratch.reshape(Vp, D)
    pooled = _make_pool(B, H, D)(x_flat, t_lin)
    return _make_head(B, D, O)(pooled, W.T, b.reshape(1, O))


# speedup vs baseline: 2.5602x; 1.1628x over previous
import functools

import jax
import jax.numpy as jnp
from jax import lax
from jax.experimental import pallas as pl
from jax.experimental.pallas import tpu as pltpu
from jax.experimental.pallas import tpu_sc as plsc

NC = 2
NS = 16
L = 16
NW = NC * NS


@functools.lru_cache(maxsize=None)
def _make_pool(B, H, D):
    assert B % NW == 0 and D % L == 0
    b_per_w = B // NW
    mesh = plsc.VectorSubcoreMesh(core_axis_name="c", subcore_axis_name="s")

    @functools.partial(
        pl.kernel,
        out_type=jax.ShapeDtypeStruct((B, D), jnp.float32),
        mesh=mesh,
        scratch_types=[
            pltpu.VMEM((b_per_w * H,), jnp.int32),
            pltpu.VMEM((H, D), jnp.float32),
            pltpu.VMEM((H, D), jnp.float32),
            pltpu.VMEM((b_per_w, D), jnp.float32),
            pltpu.SemaphoreType.DMA,
            pltpu.SemaphoreType.DMA,
        ],
        compiler_params=pltpu.CompilerParams(use_tc_tiling_on_sc=False),
    )
    def pool(x_hbm, table_hbm, out_hbm, idx_v, rows0_v, rows1_v, pooled_v,
             sem0, sem1):
        wid = lax.axis_index("s") * NC + lax.axis_index("c")
        base = wid * b_per_w
        pltpu.sync_copy(x_hbm.at[pl.ds(base * H, b_per_w * H)], idx_v)

        def gather(i, buf, sem):
            return pltpu.async_copy(
                table_hbm.at[idx_v.at[pl.ds(i * H, H)]], buf, sem
            )

        def accumulate(i, buf):
            def body(r, accs):
                return tuple(
                    a + buf[r, pl.ds(j * L, L)] for j, a in enumerate(accs)
                )

            accs = tuple(jnp.zeros((L,), jnp.float32) for _ in range(D // L))
            accs = lax.fori_loop(0, H, body, accs, unroll=4)
            for j, a in enumerate(accs):
                pooled_v[i, pl.ds(j * L, L)] = a

        gather(0, rows0_v, sem0)

        @pl.loop(0, b_per_w, step=2)
        def _pair(i):
            gather(i + 1, rows1_v, sem1)
            pltpu.make_async_copy(
                table_hbm.at[idx_v.at[pl.ds(i * H, H)]], rows0_v, sem0
            ).wait()
            accumulate(i, rows0_v)

            @pl.when(i + 2 < b_per_w)
            def _():
                gather(i + 2, rows0_v, sem0)

            pltpu.make_async_copy(
                table_hbm.at[idx_v.at[pl.ds((i + 1) * H, H)]], rows1_v, sem1
            ).wait()
            accumulate(i + 1, rows1_v)

        pltpu.sync_copy(pooled_v, out_hbm.at[pl.ds(base, b_per_w)])

    return pool


XPOSE_CB = 16384


@functools.lru_cache(maxsize=None)
def _make_xpose(V, D, CB=XPOSE_CB):
    assert 2 * D == 128
    CB2 = CB // 2
    nblk = pl.cdiv(V, CB)

    def xpose(t_ref, out_ref):
        out_ref[...] = jnp.concatenate(
            [t_ref[:, 0:CB2], t_ref[:, CB2:CB]], axis=0
        ).T

    return pl.pallas_call(
        xpose,
        grid=(nblk,),
        in_specs=[pl.BlockSpec((D, CB), lambda i: (0, i))],
        out_specs=pl.BlockSpec((CB2, 128), lambda i: (i, 0)),
        out_shape=jax.ShapeDtypeStruct((nblk * CB2, 128), jnp.float32),
    )


def _permute_idx(x, CB=XPOSE_CB):
    CB2 = CB // 2
    blk = x // CB
    r = x - blk * CB
    half = r // CB2
    p = r - half * CB2
    return blk * CB + 2 * p + half


@functools.lru_cache(maxsize=None)
def _make_head(B, D, O, BB=256):
    assert B % BB == 0

    def head(s_ref, wt_ref, b_ref, out_ref):
        logits = (
            jnp.dot(s_ref[...], wt_ref[...], preferred_element_type=jnp.float32)
            + b_ref[...]
        )
        m = jnp.max(logits, axis=-1, keepdims=True)
        e = jnp.exp(logits - m)
        out_ref[...] = e / jnp.sum(e, axis=-1, keepdims=True)

    return pl.pallas_call(
        head,
        grid=(B // BB,),
        in_specs=[
            pl.BlockSpec((BB, D), lambda i: (i, 0)),
            pl.BlockSpec((D, O), lambda i: (0, 0)),
            pl.BlockSpec((1, O), lambda i: (0, 0)),
        ],
        out_specs=pl.BlockSpec((BB, O), lambda i: (i, 0)),
        out_shape=jax.ShapeDtypeStruct((B, O), jnp.float32),
    )


def kernel(x, table, W, b):
    B, H = x.shape
    V, D = table.shape
    O = W.shape[0]
    x_flat = _permute_idx(x.reshape(B * H).astype(jnp.int32))
    scratch = _make_xpose(V, D)(table.T)
    Vp = scratch.shape[0] * scratch.shape[1] // D
    t_lin = scratch.reshape(Vp, D)
    pooled = _make_pool(B, H, D)(x_flat, t_lin)
    return _make_head(B, D, O)(pooled, W.T, b.reshape(1, O))


# speedup vs baseline: 2.9017x; 1.1334x over previous
import functools

import jax
import jax.numpy as jnp
from jax import lax
from jax.experimental import pallas as pl
from jax.experimental.pallas import tpu as pltpu
from jax.experimental.pallas import tpu_sc as plsc

NC = 2
NS = 16
L = 16
NW = NC * NS


NBUF = 4


@functools.lru_cache(maxsize=None)
def _make_pool(B, H, D):
    assert B % NW == 0 and D % L == 0
    b_per_w = B // NW
    assert b_per_w % NBUF == 0
    mesh = plsc.VectorSubcoreMesh(core_axis_name="c", subcore_axis_name="s")

    @functools.partial(
        pl.kernel,
        out_type=jax.ShapeDtypeStruct((B, D), jnp.float32),
        mesh=mesh,
        scratch_types=[
            pltpu.VMEM((b_per_w * H,), jnp.int32),
            [pltpu.VMEM((H, D), jnp.float32) for _ in range(NBUF)],
            pltpu.VMEM((b_per_w, D), jnp.float32),
            [pltpu.SemaphoreType.DMA for _ in range(NBUF)],
        ],
        compiler_params=pltpu.CompilerParams(use_tc_tiling_on_sc=False),
    )
    def pool(x_hbm, table_hbm, out_hbm, idx_v, rows, pooled_v, sems):
        wid = lax.axis_index("s") * NC + lax.axis_index("c")
        base = wid * b_per_w
        pltpu.sync_copy(x_hbm.at[pl.ds(base * H, b_per_w * H)], idx_v)

        def gather(i, k):
            return pltpu.async_copy(
                table_hbm.at[idx_v.at[pl.ds(i * H, H)]], rows[k], sems[k]
            )

        def accumulate(i, k):
            buf = rows[k]

            def body(r, accs):
                return tuple(
                    a + buf[r, pl.ds(j * L, L)] for j, a in enumerate(accs)
                )

            accs = tuple(jnp.zeros((L,), jnp.float32) for _ in range(D // L))
            accs = lax.fori_loop(0, H, body, accs, unroll=4)
            for j, a in enumerate(accs):
                pooled_v[i, pl.ds(j * L, L)] = a

        for k in range(NBUF - 1):
            gather(k, k)

        @pl.loop(0, b_per_w, step=NBUF)
        def _group(i):
            for k in range(NBUF):
                nxt = i + k + NBUF - 1

                @pl.when(nxt < b_per_w)
                def _():
                    gather(nxt, (k + NBUF - 1) % NBUF)

                pltpu.make_async_copy(
                    table_hbm.at[idx_v.at[pl.ds((i + k) * H, H)]],
                    rows[k],
                    sems[k],
                ).wait()
                accumulate(i + k, k)

        pltpu.sync_copy(pooled_v, out_hbm.at[pl.ds(base, b_per_w)])

    return pool


XPOSE_CB = 16384


@functools.lru_cache(maxsize=None)
def _make_xpose(V, D, CB=XPOSE_CB):
    assert 2 * D == 128
    CB2 = CB // 2
    nblk = pl.cdiv(V, CB)

    def xpose(t_ref, out_ref):
        out_ref[...] = jnp.concatenate(
            [t_ref[:, 0:CB2], t_ref[:, CB2:CB]], axis=0
        ).T

    return pl.pallas_call(
        xpose,
        grid=(nblk,),
        in_specs=[pl.BlockSpec((D, CB), lambda i: (0, i))],
        out_specs=pl.BlockSpec((CB2, 128), lambda i: (i, 0)),
        out_shape=jax.ShapeDtypeStruct((nblk * CB2, 128), jnp.float32),
    )


def _permute_idx(x, CB=XPOSE_CB):
    CB2 = CB // 2
    blk = x // CB
    r = x - blk * CB
    half = r // CB2
    p = r - half * CB2
    return blk * CB + 2 * p + half


@functools.lru_cache(maxsize=None)
def _make_head(B, D, O, BB=256):
    assert B % BB == 0

    def head(s_ref, wt_ref, b_ref, out_ref):
        logits = (
            jnp.dot(s_ref[...], wt_ref[...], preferred_element_type=jnp.float32)
            + b_ref[...]
        )
        m = jnp.max(logits, axis=-1, keepdims=True)
        e = jnp.exp(logits - m)
        out_ref[...] = e / jnp.sum(e, axis=-1, keepdims=True)

    return pl.pallas_call(
        head,
        grid=(B // BB,),
        in_specs=[
            pl.BlockSpec((BB, D), lambda i: (i, 0)),
            pl.BlockSpec((D, O), lambda i: (0, 0)),
            pl.BlockSpec((1, O), lambda i: (0, 0)),
        ],
        out_specs=pl.BlockSpec((BB, O), lambda i: (i, 0)),
        out_shape=jax.ShapeDtypeStruct((B, O), jnp.float32),
    )


def kernel(x, table, W, b):
    B, H = x.shape
    V, D = table.shape
    O = W.shape[0]
    x_flat = _permute_idx(x.reshape(B * H).astype(jnp.int32))
    scratch = _make_xpose(V, D)(table.T)
    Vp = scratch.shape[0] * scratch.shape[1] // D
    t_lin = scratch.reshape(Vp, D)
    pooled = _make_pool(B, H, D)(x_flat, t_lin)
    return _make_head(B, D, O)(pooled, W.T, b.reshape(1, O))


# speedup vs baseline: 3.0501x; 1.0511x over previous
import functools

import jax
import jax.numpy as jnp
from jax import lax
from jax.experimental import pallas as pl
from jax.experimental.pallas import tpu as pltpu
from jax.experimental.pallas import tpu_sc as plsc

NC = 2
NS = 16
L = 16
NW = NC * NS


NBUF = 4


@functools.lru_cache(maxsize=None)
def _make_pool(B, H, D):
    assert B % NW == 0 and D % L == 0
    b_per_w = B // NW
    assert b_per_w % NBUF == 0
    mesh = plsc.VectorSubcoreMesh(core_axis_name="c", subcore_axis_name="s")

    @functools.partial(
        pl.kernel,
        out_type=jax.ShapeDtypeStruct((B, D), jnp.float32),
        mesh=mesh,
        scratch_types=[
            pltpu.VMEM((b_per_w * H,), jnp.int32),
            [pltpu.VMEM((H, D), jnp.float32) for _ in range(NBUF)],
            pltpu.VMEM((b_per_w, D), jnp.float32),
            [pltpu.SemaphoreType.DMA for _ in range(NBUF)],
        ],
        compiler_params=pltpu.CompilerParams(use_tc_tiling_on_sc=False),
    )
    def pool(x_hbm, table_hbm, out_hbm, idx_v, rows, pooled_v, sems):
        wid = lax.axis_index("s") * NC + lax.axis_index("c")
        base = wid * b_per_w
        pltpu.sync_copy(x_hbm.at[pl.ds(base * H, b_per_w * H)], idx_v)

        def gather(i, k):
            return pltpu.async_copy(
                table_hbm.at[idx_v.at[pl.ds(i * H, H)]], rows[k], sems[k]
            )

        def accumulate(i, k):
            buf = rows[k]

            def body(r, accs):
                return tuple(
                    a + buf[r, pl.ds(j * L, L)] for j, a in enumerate(accs)
                )

            accs = tuple(jnp.zeros((L,), jnp.float32) for _ in range(D // L))
            accs = lax.fori_loop(0, H, body, accs, unroll=4)
            for j, a in enumerate(accs):
                pooled_v[i, pl.ds(j * L, L)] = a

        for k in range(NBUF - 1):
            gather(k, k)

        @pl.loop(0, b_per_w, step=NBUF)
        def _group(i):
            for k in range(NBUF):
                nxt = i + k + NBUF - 1

                @pl.when(nxt < b_per_w)
                def _():
                    gather(nxt, (k + NBUF - 1) % NBUF)

                pltpu.make_async_copy(
                    table_hbm.at[idx_v.at[pl.ds((i + k) * H, H)]],
                    rows[k],
                    sems[k],
                ).wait()
                accumulate(i + k, k)

        pltpu.sync_copy(pooled_v, out_hbm.at[pl.ds(base, b_per_w)])

    return pool


XPOSE_CB = 16384


@functools.lru_cache(maxsize=None)
def _make_xpose(V, D, CB=XPOSE_CB):
    assert 2 * D == 128
    CB2 = CB // 2
    nblk = pl.cdiv(V, CB)

    def xpose(t_ref, out_ref):
        out_ref[...] = jnp.concatenate(
            [t_ref[:, 0:CB2], t_ref[:, CB2:CB]], axis=0
        ).T

    return pl.pallas_call(
        xpose,
        grid=(nblk,),
        in_specs=[pl.BlockSpec((D, CB), lambda i: (0, i))],
        out_specs=pl.BlockSpec((CB2, 128), lambda i: (i, 0)),
        out_shape=jax.ShapeDtypeStruct((nblk * CB2, 128), jnp.float32),
    )


def _permute_idx(x, CB=XPOSE_CB):
    CB2 = CB // 2
    blk = x // CB
    r = x - blk * CB
    half = r // CB2
    p = r - half * CB2
    return blk * CB + 2 * p + half


@functools.lru_cache(maxsize=None)
def _make_head(B, D, O, BB=256):
    assert B % BB == 0

    def head(s_ref, w_ref, b_ref, out_ref):
        logits = (
            jnp.dot(w_ref[...], s_ref[...].T, preferred_element_type=jnp.float32)
            + b_ref[...]
        )
        m = jnp.max(logits, axis=0, keepdims=True)
        e = jnp.exp(logits - m)
        out_ref[...] = e / jnp.sum(e, axis=0, keepdims=True)

    return pl.pallas_call(
        head,
        grid=(B // BB,),
        in_specs=[
            pl.BlockSpec((BB, D), lambda i: (i, 0)),
            pl.BlockSpec((O, D), lambda i: (0, 0)),
            pl.BlockSpec((O, 1), lambda i: (0, 0)),
        ],
        out_specs=pl.BlockSpec((O, BB), lambda i: (0, i)),
        out_shape=jax.ShapeDtypeStruct((O, B), jnp.float32),
    )


def kernel(x, table, W, b):
    B, H = x.shape
    V, D = table.shape
    O = W.shape[0]
    x_flat = _permute_idx(x.reshape(B * H).astype(jnp.int32))
    scratch = _make_xpose(V, D)(table.T)
    Vp = scratch.shape[0] * scratch.shape[1] // D
    t_lin = scratch.reshape(Vp, D)
    pooled = _make_pool(B, H, D)(x_flat, t_lin)
    return _make_head(B, D, O)(pooled, W, b.reshape(O, 1)).T
